# Initial kernel scaffold; baseline (speedup 1.0000x reference)
import functools

import jax
import jax.numpy as jnp
from jax import lax
from jax.experimental import pallas as pl
from jax.experimental.pallas import tpu as pltpu
from jax.experimental.pallas import tpu_sc as plsc

N_NODES = 10000
N_EDGES = 320000
D_FEAT = 128
L = 16

NC = 2
NS = 16
NW = NC * NS
EPW = N_EDGES // NW
B = 80
NCH = EPW // B

_mesh = plsc.VectorSubcoreMesh(core_axis_name="c", subcore_axis_name="s")


@functools.partial(
    pl.kernel,
    mesh=_mesh,
    out_type=jax.ShapeDtypeStruct((N_EDGES,), jnp.float32),
    scratch_types=[
        pltpu.VMEM((EPW,), jnp.int32),
        pltpu.VMEM((EPW,), jnp.int32),
        pltpu.VMEM((B, D_FEAT), jnp.float32),
        pltpu.VMEM((B, D_FEAT), jnp.float32),
        pltpu.VMEM((EPW,), jnp.float32),
        pltpu.SemaphoreType.DMA,
        pltpu.SemaphoreType.DMA,
    ],
)
def _edge_dot(src_hbm, dst_hbm, feat_hbm, out_hbm,
              src_v, dst_v, srows, drows, out_v, sem_s, sem_d):
    wid = lax.axis_index("s") * NC + lax.axis_index("c")
    base = wid * EPW

    pltpu.sync_copy(src_hbm.at[pl.ds(base, EPW)], src_v)
    pltpu.sync_copy(dst_hbm.at[pl.ds(base, EPW)], dst_v)

    def chunk_body(c, carry):
        off = c * B
        cp_s = pltpu.async_copy(
            feat_hbm.at[src_v.at[pl.ds(off, B)]], srows, sem_s)
        cp_d = pltpu.async_copy(
            feat_hbm.at[dst_v.at[pl.ds(off, B)]], drows, sem_d)
        cp_s.wait()
        cp_d.wait()

        def edge_body(e, carry2):
            acc = jnp.zeros((L,), jnp.float32)
            for db in range(D_FEAT // L):
                s = srows[e, pl.ds(db * L, L)]
                t = drows[e, pl.ds(db * L, L)]
                acc = acc + s * t
            out_v[off + e] = jnp.sum(acc)
            return carry2

        lax.fori_loop(0, B, edge_body, 0)
        return carry

    lax.fori_loop(0, NCH, chunk_body, 0)
    pltpu.sync_copy(out_v, out_hbm.at[pl.ds(base, EPW)])


def kernel(edges, feat):
    src = edges[:, 0].astype(jnp.int32)
    dst = edges[:, 1].astype(jnp.int32)
    out = _edge_dot(src, dst, feat.astype(jnp.float32))
    return out[:, None]


# speedup vs baseline: 4.4742x; 4.4742x over previous
import functools

import jax
import jax.numpy as jnp
from jax import lax
from jax.experimental import pallas as pl
from jax.experimental.pallas import tpu as pltpu
from jax.experimental.pallas import tpu_sc as plsc

N_NODES = 10000
N_EDGES = 320000
D_FEAT = 128
L = 16

NC = 2
NS = 16
NW = NC * NS
EPW = N_EDGES // NW
B = 80
NCH = EPW // B

_mesh = plsc.VectorSubcoreMesh(core_axis_name="c", subcore_axis_name="s")


@functools.partial(
    pl.kernel,
    mesh=_mesh,
    compiler_params=pltpu.CompilerParams(needs_layout_passes=False),
    out_type=jax.ShapeDtypeStruct((N_EDGES,), jnp.float32),
    scratch_types=[
        pltpu.VMEM((EPW,), jnp.int32),
        pltpu.VMEM((EPW,), jnp.int32),
        pltpu.VMEM((B, D_FEAT), jnp.float32),
        pltpu.VMEM((B, D_FEAT), jnp.float32),
        pltpu.VMEM((B * L,), jnp.float32),
        pltpu.VMEM((EPW,), jnp.float32),
        pltpu.SemaphoreType.DMA,
        pltpu.SemaphoreType.DMA,
    ],
)
def _edge_dot(src_hbm, dst_hbm, feat_hbm, out_hbm,
              src_v, dst_v, srows, drows, acc_v, out_v, sem_s, sem_d):
    wid = lax.axis_index("s") * NC + lax.axis_index("c")
    base = wid * EPW

    pltpu.sync_copy(src_hbm.at[pl.ds(base, EPW)], src_v)
    pltpu.sync_copy(dst_hbm.at[pl.ds(base, EPW)], dst_v)

    def chunk_body(c, carry):
        off = c * B
        cp_s = pltpu.async_copy(
            feat_hbm.at[src_v.at[pl.ds(off, B)]], srows, sem_s)
        cp_d = pltpu.async_copy(
            feat_hbm.at[dst_v.at[pl.ds(off, B)]], drows, sem_d)
        cp_s.wait()
        cp_d.wait()

        def edge_body(e, carry2):
            acc = jnp.zeros((L,), jnp.float32)
            for db in range(D_FEAT // L):
                s = srows[e, pl.ds(db * L, L)]
                t = drows[e, pl.ds(db * L, L)]
                acc = acc + s * t
            acc_v[pl.ds(e * L, L)] = acc
            return carry2

        lax.fori_loop(0, B, edge_body, 0)

        def group_body(g, carry2):
            rows = g * L + lax.iota(jnp.int32, L)
            tot = jnp.zeros((L,), jnp.float32)
            for j in range(L):
                tot = tot + plsc.load_gather(acc_v, [rows * L + j])
            out_v[pl.ds(off + g * L, L)] = tot
            return carry2

        lax.fori_loop(0, B // L, group_body, 0)
        return carry

    lax.fori_loop(0, NCH, chunk_body, 0)
    pltpu.sync_copy(out_v, out_hbm.at[pl.ds(base, EPW)])


def kernel(edges, feat):
    src = edges[:, 0].astype(jnp.int32)
    dst = edges[:, 1].astype(jnp.int32)
    out = _edge_dot(src, dst, feat.astype(jnp.float32))
    return out[:, None]
